# baseline (speedup 1.0000x reference)
import functools

import jax
import jax.numpy as jnp
from jax import lax
from jax.experimental import pallas as pl
from jax.experimental.pallas import tpu as pltpu
from jax.experimental.pallas import tpu_sc as plsc

NCLS = 16
BATCH = 16384
_L = 16


def _argmax_body(x_ref, o_ref):
    x = x_ref[...]
    m = jnp.max(x, axis=1, keepdims=True)
    ii = lax.broadcasted_iota(jnp.int32, x.shape, 1)
    cand = jnp.where(x == m, ii, jnp.int32(x.shape[1]))
    o_ref[...] = jnp.min(cand, axis=1, keepdims=True)


def _compact_body(pred_hbm, out_hbm, pred_v, out_v, sem1):
    wid = lax.axis_index("s") * 2 + lax.axis_index("c")

    @pl.when(wid < NCLS)
    def _():
        cls = wid
        cp1 = pltpu.async_copy(pred_hbm, pred_v, sem1)
        neg1 = jnp.full((_L,), -1, jnp.int32)

        def fill(i, carry):
            out_v[pl.ds(i * _L, _L)] = neg1
            return carry

        lax.fori_loop(0, BATCH // _L, fill, 0)
        cp1.wait()

        cls_v = jnp.full((_L,), cls, jnp.int32)
        lane = lax.iota(jnp.int32, _L)
        ones = jnp.full((_L,), 1, jnp.int32)
        zeros = jnp.full((_L,), 0, jnp.int32)

        def body(g, ptr):
            v = pred_v[pl.ds(g * _L, _L)]
            mask = v == cls_v
            m32 = jnp.where(mask, ones, zeros)
            inc = plsc.cumsum(m32)
            idx = lane + jnp.full((_L,), g * _L, jnp.int32)
            pos = jnp.full((_L,), ptr, jnp.int32) + inc - m32
            plsc.store_scatter(out_v, [pos], idx, mask=mask)
            return ptr + jnp.sum(m32)

        lax.fori_loop(0, BATCH // _L, body, jnp.int32(0))
        pltpu.sync_copy(out_v.at[pl.ds(0, BATCH)], out_hbm.at[cls])


def kernel(predictions):
    pred = pl.pallas_call(
        _argmax_body,
        out_shape=jax.ShapeDtypeStruct((BATCH, 1), jnp.int32),
    )(predictions).reshape(BATCH)

    mesh = plsc.VectorSubcoreMesh(core_axis_name="c", subcore_axis_name="s")
    compact = pl.kernel(
        _compact_body,
        out_type=jax.ShapeDtypeStruct((NCLS, BATCH), jnp.int32),
        mesh=mesh,
        compiler_params=pltpu.CompilerParams(needs_layout_passes=False),
        scratch_types=[
            pltpu.VMEM((BATCH,), jnp.int32),
            pltpu.VMEM((BATCH + _L,), jnp.int32),
            pltpu.SemaphoreType.DMA,
        ],
    )
    out = compact(pred)
    return out.astype(jnp.int64)


# speedup vs baseline: 1.6070x; 1.6070x over previous
import functools

import jax
import jax.numpy as jnp
from jax import lax
from jax.experimental import pallas as pl
from jax.experimental.pallas import tpu as pltpu
from jax.experimental.pallas import tpu_sc as plsc

NCLS = 16
BATCH = 16384
_L = 16


def _argmax_body(x_ref, o_ref):
    x = x_ref[...]
    m = jnp.max(x, axis=0, keepdims=True)
    ii = lax.broadcasted_iota(jnp.int32, x.shape, 0)
    cand = jnp.where(x == m, ii, jnp.int32(x.shape[0]))
    o_ref[...] = jnp.min(cand, axis=0)


def _compact_body(pred_hbm, out_hbm, pred_v, out_v, sem1):
    wid = lax.axis_index("s") * 2 + lax.axis_index("c")

    @pl.when(wid < NCLS)
    def _():
        cls = wid
        cp1 = pltpu.async_copy(pred_hbm, pred_v, sem1)
        neg1 = jnp.full((_L,), -1, jnp.int32)

        def fill(i, carry):
            out_v[pl.ds(i * _L, _L)] = neg1
            return carry

        lax.fori_loop(0, BATCH // _L, fill, 0)
        cp1.wait()

        cls_v = jnp.full((_L,), cls, jnp.int32)
        lane = lax.iota(jnp.int32, _L)
        ones = jnp.full((_L,), 1, jnp.int32)
        zeros = jnp.full((_L,), 0, jnp.int32)

        def body(g, ptr):
            v = pred_v[pl.ds(g * _L, _L)]
            mask = v == cls_v
            m32 = jnp.where(mask, ones, zeros)
            inc = plsc.cumsum(m32)
            idx = lane + jnp.full((_L,), g * _L, jnp.int32)
            pos = jnp.full((_L,), ptr, jnp.int32) + inc - m32
            plsc.store_scatter(out_v, [pos], idx, mask=mask)
            return ptr + jnp.sum(m32)

        lax.fori_loop(0, BATCH // _L, body, jnp.int32(0))
        pltpu.sync_copy(out_v.at[pl.ds(0, BATCH)], out_hbm.at[cls])


def kernel(predictions):
    pred = pl.pallas_call(
        _argmax_body,
        out_shape=jax.ShapeDtypeStruct((BATCH,), jnp.int32),
    )(predictions.T)

    mesh = plsc.VectorSubcoreMesh(core_axis_name="c", subcore_axis_name="s")
    compact = pl.kernel(
        _compact_body,
        out_type=jax.ShapeDtypeStruct((NCLS, BATCH), jnp.int32),
        mesh=mesh,
        compiler_params=pltpu.CompilerParams(needs_layout_passes=False),
        scratch_types=[
            pltpu.VMEM((BATCH,), jnp.int32),
            pltpu.VMEM((BATCH + _L,), jnp.int32),
            pltpu.SemaphoreType.DMA,
        ],
    )
    out = compact(pred)
    return out.astype(jnp.int64)
